# baseline (speedup 1.0000x reference)
import math

import jax
import jax.numpy as jnp
from jax import lax
from jax.experimental import pallas as pl
from jax.experimental.pallas import tpu as pltpu
from jax.experimental.pallas import tpu_sc as plsc

N = 10000
E = 160000
D = 128
H = 64
NH = 4
NC = 256
K = 50

NWORK = 32
CH = 128
CPW = 40
E2 = NWORK * CPW * CH
NCHUNK = E2 // CH
NTILE = 16
TPN = N // NTILE
TPN_C = 125

BETA = (2.0 / K * (1.0 - math.exp(-5.0))) ** (-2.0)

BE = 1024
BE2 = 1024
BN = 1000

_mesh_cache = []


def _mesh():
    if not _mesh_cache:
        _mesh_cache.append(
            plsc.VectorSubcoreMesh(core_axis_name="c", subcore_axis_name="s"))
    return _mesh_cache[0]


f32 = jnp.float32


bf16 = jnp.bfloat16


def _p1_body(hx_hbm, src_hbm, dst_hbm, hs_out, hd_out,
             sidx, didx, bs0, bs1, bd0, bd1,
             g0s, g1s, g2s, g3s, w0s, w1s, w2s, w3s):
    c = lax.axis_index("c")
    s = lax.axis_index("s")
    wid = s * 2 + c
    g0 = wid * CPW
    pltpu.sync_copy(src_hbm.at[pl.ds(g0, CPW)], sidx)
    pltpu.sync_copy(dst_hbm.at[pl.ds(g0, CPW)], didx)

    def body(j, carry):
        gg = g0 + 2 * j
        c0 = pltpu.async_copy(hx_hbm.at[sidx.at[2 * j]], bs0, g0s)
        c1 = pltpu.async_copy(hx_hbm.at[didx.at[2 * j]], bd0, g1s)
        c2 = pltpu.async_copy(hx_hbm.at[sidx.at[2 * j + 1]], bs1, g2s)
        c3 = pltpu.async_copy(hx_hbm.at[didx.at[2 * j + 1]], bd1, g3s)
        c0.wait()
        w0 = pltpu.async_copy(bs0, hs_out.at[pl.ds(gg * CH, CH)], w0s)
        c1.wait()
        w1 = pltpu.async_copy(bd0, hd_out.at[pl.ds(gg * CH, CH)], w1s)
        c2.wait()
        w2 = pltpu.async_copy(bs1, hs_out.at[pl.ds((gg + 1) * CH, CH)], w2s)
        c3.wait()
        w3 = pltpu.async_copy(bd1, hd_out.at[pl.ds((gg + 1) * CH, CH)], w3s)
        w0.wait()
        w1.wait()
        w2.wait()
        w3.wait()
        return carry

    lax.fori_loop(0, CPW // 2, body, 0)


def _p1(hx, src2, dst2):
    return pl.kernel(
        _p1_body,
        out_type=(
            jax.ShapeDtypeStruct((E2, 80), f32),
            jax.ShapeDtypeStruct((E2, 80), f32),
        ),
        mesh=_mesh(),
        compiler_params=pltpu.CompilerParams(use_tc_tiling_on_sc=False),
        scratch_types=[
            pltpu.VMEM((CPW, CH), jnp.int32),
            pltpu.VMEM((CPW, CH), jnp.int32),
            pltpu.VMEM((CH, 80), f32),
            pltpu.VMEM((CH, 80), f32),
            pltpu.VMEM((CH, 80), f32),
            pltpu.VMEM((CH, 80), f32),
        ] + [pltpu.SemaphoreType.DMA] * 8,
    )(hx, src2, dst2)


def _p2_body(hs_r, hd_r, Wina, Winb, bin_, means, W1a, W1b, W1fx,
             W1dn, bo1, Wo2, bo2, Watt, batt, he_o, wc_o, xn_o):
    i = pl.program_id(0)
    hxs = hs_r[...]
    hxd = hd_r[...]
    hsu = lax.bitcast_convert_type(hxs[:, :D // 2], jnp.uint32)
    hdu = lax.bitcast_convert_type(hxd[:, :D // 2], jnp.uint32)
    hse = lax.bitcast_convert_type(hsu << 16, f32).astype(bf16)
    hso = lax.bitcast_convert_type(hsu & jnp.uint32(0xFFFF0000),
                                   f32).astype(bf16)
    hde = lax.bitcast_convert_type(hdu << 16, f32).astype(bf16)
    hdo = lax.bitcast_convert_type(hdu & jnp.uint32(0xFFFF0000),
                                   f32).astype(bf16)
    dx = hxs[:, D // 2:] - hxd[:, D // 2:]
    dn = jnp.sqrt(jnp.sum(dx * dx, axis=-1, keepdims=True) + 1e-14)
    wa, wb = Wina[...], Winb[...]
    h1 = (jnp.dot(hse, wa[:D // 2], preferred_element_type=f32)
          + jnp.dot(hso, wa[D // 2:], preferred_element_type=f32)
          + jnp.dot(hde, wb[:D // 2], preferred_element_type=f32)
          + jnp.dot(hdo, wb[D // 2:], preferred_element_type=f32)
          + bin_[...])
    expn = jnp.exp(-BETA * (jnp.exp(-dn) - means[...]) ** 2)
    fx = expn * h1
    wc1, wd1 = W1a[...], W1b[...]
    t = (jnp.dot(hse, wc1[:D // 2], preferred_element_type=f32)
         + jnp.dot(hso, wc1[D // 2:], preferred_element_type=f32)
         + jnp.dot(hde, wd1[:D // 2], preferred_element_type=f32)
         + jnp.dot(hdo, wd1[D // 2:], preferred_element_type=f32)
         + fx @ W1fx[...] + dn * W1dn[...] + bo1[...])
    t = t * jax.nn.sigmoid(t)
    he = t @ Wo2[...] + bo2[...]
    att = he @ Watt[...] + batt[...]
    cel = jnp.where(att > 0, att, 2.0 * (jnp.exp(att * 0.5) - 1.0))
    w = jnp.exp(cel)
    ci = lax.broadcasted_iota(jnp.int32, (BE, 16), 1)
    rowid = i * BE + lax.broadcasted_iota(jnp.int32, (BE, 1), 0)
    valid = (rowid < E).astype(f32)
    wc = (jnp.where(ci < 4, w, 0.0) + jnp.where(ci == 4, 1.0, 0.0)) * valid
    he_o[...] = he
    wc_o[...] = wc
    xn_o[...] = dx / (dn + 1e-5)


def _p2(hxs, hxd, Wina, Winb, bin_, means, W1a, W1b, W1fx, W1dn,
        bo1, Wo2, bo2, Watt, batt):
    wspec = lambda shp: pl.BlockSpec(shp, lambda i: (0, 0))
    return pl.pallas_call(
        _p2_body,
        grid=(E2 // BE,),
        in_specs=[
            pl.BlockSpec((BE, 80), lambda i: (i, 0)),
            pl.BlockSpec((BE, 80), lambda i: (i, 0)),
            wspec((D, K)), wspec((D, K)), wspec((1, K)), wspec((1, K)),
            wspec((D, H)), wspec((D, H)), wspec((K, H)), wspec((1, H)),
            wspec((1, H)), wspec((H, H)), wspec((1, H)),
            wspec((H, 16)), wspec((1, 16)),
        ],
        out_specs=[
            pl.BlockSpec((BE, H), lambda i: (i, 0)),
            pl.BlockSpec((BE, 16), lambda i: (i, 0)),
            pl.BlockSpec((BE, 16), lambda i: (i, 0)),
        ],
        out_shape=(
            jax.ShapeDtypeStruct((E2, H), f32),
            jax.ShapeDtypeStruct((E2, 16), f32),
            jax.ShapeDtypeStruct((E2, 16), f32),
        ),
    )(hxs, hxd, Wina, Winb, bin_, means, W1a, W1b, W1fx, W1dn, bo1,
      Wo2, bo2, Watt, batt)


def _p3_body(wc_hbm, dst_hbm, z16_hbm, sa_out, sb_out, acc, idx, wbuf, zv):
    c = lax.axis_index("c")
    s = lax.axis_index("s")
    pltpu.sync_copy(z16_hbm, zv)
    r0 = s * TPN
    for k in range(5):
        pltpu.sync_copy(zv.at[pl.ds(0, TPN_C)],
                        acc.at[pl.ds(r0 + k * TPN_C, TPN_C)])
    plsc.subcore_barrier()
    g0 = (c * NTILE + s) * CPW
    pltpu.sync_copy(dst_hbm.at[pl.ds(g0, CPW)], idx)

    def body(g, carry):
        row0 = (g0 + g) * CH
        pltpu.sync_copy(wc_hbm.at[pl.ds(row0, CH)], wbuf)
        pltpu.sync_copy(wbuf, acc.at[idx.at[g]], add=True)
        return carry

    lax.fori_loop(0, CPW, body, 0)
    plsc.subcore_barrier()
    for k in range(5):
        rr = r0 + k * TPN_C

        @pl.when(c == 0)
        def _():
            pltpu.sync_copy(acc.at[pl.ds(rr, TPN_C)],
                            sa_out.at[pl.ds(rr, TPN_C)])

        @pl.when(c == 1)
        def _():
            pltpu.sync_copy(acc.at[pl.ds(rr, TPN_C)],
                            sb_out.at[pl.ds(rr, TPN_C)])


def _p3(wc, dst2, z16):
    return pl.kernel(
        _p3_body,
        out_type=(
            jax.ShapeDtypeStruct((N, 16), f32),
            jax.ShapeDtypeStruct((N, 16), f32),
        ),
        mesh=_mesh(),
        compiler_params=pltpu.CompilerParams(use_tc_tiling_on_sc=False),
        scratch_types=[
            pltpu.VMEM_SHARED((N, 16), f32),
            pltpu.VMEM((CPW, CH), jnp.int32),
            pltpu.VMEM((CH, 16), f32),
            pltpu.VMEM((CH, 16), f32),
        ],
    )(wc, dst2, z16)


def _p3b_body(sa_hbm, sb_hbm, dst_hbm, ga_out, gb_out, idx, a0, b0, a1, b1,
              ga0, gb0, ga1, gb1, wa0, wb0, wa1, wb1):
    c = lax.axis_index("c")
    s = lax.axis_index("s")
    wid = s * 2 + c
    g0 = wid * CPW
    pltpu.sync_copy(dst_hbm.at[pl.ds(g0, CPW)], idx)

    def body(j, carry):
        gg = g0 + 2 * j
        cpa0 = pltpu.async_copy(sa_hbm.at[idx.at[2 * j]], a0, ga0)
        cpb0 = pltpu.async_copy(sb_hbm.at[idx.at[2 * j]], b0, gb0)
        cpa1 = pltpu.async_copy(sa_hbm.at[idx.at[2 * j + 1]], a1, ga1)
        cpb1 = pltpu.async_copy(sb_hbm.at[idx.at[2 * j + 1]], b1, gb1)
        cpa0.wait()
        w0 = pltpu.async_copy(a0, ga_out.at[pl.ds(gg * CH, CH)], wa0)
        cpb0.wait()
        w1 = pltpu.async_copy(b0, gb_out.at[pl.ds(gg * CH, CH)], wb0)
        cpa1.wait()
        w2 = pltpu.async_copy(a1, ga_out.at[pl.ds((gg + 1) * CH, CH)], wa1)
        cpb1.wait()
        w3 = pltpu.async_copy(b1, gb_out.at[pl.ds((gg + 1) * CH, CH)], wb1)
        w0.wait()
        w1.wait()
        w2.wait()
        w3.wait()
        return carry

    lax.fori_loop(0, CPW // 2, body, 0)


def _p3b(sa, sb, dst2):
    return pl.kernel(
        _p3b_body,
        out_type=(
            jax.ShapeDtypeStruct((E2, 16), f32),
            jax.ShapeDtypeStruct((E2, 16), f32),
        ),
        mesh=_mesh(),
        compiler_params=pltpu.CompilerParams(use_tc_tiling_on_sc=False),
        scratch_types=[
            pltpu.VMEM((CPW, CH), jnp.int32),
            pltpu.VMEM((CH, 16), f32),
            pltpu.VMEM((CH, 16), f32),
            pltpu.VMEM((CH, 16), f32),
            pltpu.VMEM((CH, 16), f32),
        ] + [pltpu.SemaphoreType.DMA] * 8,
    )(sa, sb, dst2)


def _p4_body(he_r, wc_r, xn_r, ga_r, gb_r, Wx, o0, o1, o2, o3, o4, o5, o6, o7):
    i = pl.program_id(0)
    s = ga_r[...] + gb_r[...]
    catt = wc_r[...][:, 0:4] / (s[:, 0:4] + 1e-16)
    he = he_r[...]
    hea = jnp.concatenate([he * catt[:, k:k + 1] for k in range(4)], axis=1)
    coeff = jnp.tanh(jnp.dot(hea.astype(bf16), Wx[...],
                             preferred_element_type=f32))
    rowid = i * BE2 + lax.broadcasted_iota(jnp.int32, (BE2, 1), 0)
    valid = (rowid < E).astype(f32)
    xn = xn_r[...]
    heam = hea * valid
    cx = coeff * xn[:, 0:1] * valid
    cy = coeff * xn[:, 1:2] * valid
    cz = coeff * xn[:, 2:3] * valid
    o0[...] = heam[:, :D]
    o1[...] = heam[:, D:]
    o2[...] = cx[:, :D]
    o3[...] = cx[:, D:]
    o4[...] = cy[:, :D]
    o5[...] = cy[:, D:]
    o6[...] = cz[:, :D]
    o7[...] = cz[:, D:]


def _p4(he, wc, xn, ga, gb, Wx):
    espec = pl.BlockSpec((BE2, 16), lambda i: (i, 0))
    ospec = pl.BlockSpec((BE2, D), lambda i: (i, 0))
    return pl.pallas_call(
        _p4_body,
        grid=(E2 // BE2,),
        in_specs=[
            pl.BlockSpec((BE2, H), lambda i: (i, 0)),
            espec, espec, espec, espec,
            pl.BlockSpec((NC, NC), lambda i: (0, 0)),
        ],
        out_specs=[ospec] * 8,
        out_shape=tuple(jax.ShapeDtypeStruct((E2, D), f32) for _ in range(8)),
    )(he, wc, xn, ga, gb, Wx)


def _p5_body(e0, e1, e2, e3, e4, e5, e6, e7, dst_hbm, z128_hbm, out_hbm,
             acc, idx, db0, db1, l0, l1, s0, s1):
    c = lax.axis_index("c")
    s = lax.axis_index("s")
    g0 = (c * NTILE + s) * CPW
    pltpu.sync_copy(dst_hbm.at[pl.ds(g0, CPW)], idx)
    r0 = s * TPN
    for p, e_hbm in enumerate([e0, e1, e2, e3, e4, e5, e6, e7]):
        for k in range(5):
            pltpu.sync_copy(z128_hbm.at[pl.ds(0, TPN_C)],
                            acc.at[pl.ds(r0 + k * TPN_C, TPN_C)])
        plsc.subcore_barrier()

        def body(j, carry):
            gg = g0 + 2 * j
            cl0 = pltpu.async_copy(e_hbm.at[pl.ds(gg * CH, CH)], db0, l0)
            cl1 = pltpu.async_copy(e_hbm.at[pl.ds((gg + 1) * CH, CH)], db1, l1)
            cl0.wait()
            cs0 = pltpu.async_copy(db0, acc.at[idx.at[2 * j]], s0, add=True)
            cl1.wait()
            cs1 = pltpu.async_copy(db1, acc.at[idx.at[2 * j + 1]], s1,
                                   add=True)
            cs0.wait()
            cs1.wait()
            return carry

        lax.fori_loop(0, CPW // 2, body, 0)
        plsc.subcore_barrier()
        for k in range(5):
            rr = r0 + k * TPN_C
            pltpu.sync_copy(acc.at[pl.ds(rr, TPN_C)],
                            out_hbm.at[p, c, pl.ds(rr, TPN_C)])
        plsc.subcore_barrier()


def _p5(edats, dst2, z128):
    return pl.kernel(
        _p5_body,
        out_type=jax.ShapeDtypeStruct((8, 2, N, D), f32),
        mesh=_mesh(),
        compiler_params=pltpu.CompilerParams(use_tc_tiling_on_sc=False),
        scratch_types=[
            pltpu.VMEM_SHARED((N, D), f32),
            pltpu.VMEM((CPW, CH), jnp.int32),
            pltpu.VMEM((CH, D), f32),
            pltpu.VMEM((CH, D), f32),
        ] + [pltpu.SemaphoreType.DMA] * 4,
    )(*edats, dst2, z128)


def _p6_body(h_r, xp_r, vp_r, sa_r, sb_r, acc_r, WpnL, WpnR, bpn1, Wpn2, bpn2,
             W1h, W1eL, W1eR, W1hc, bn1, Wn2, bn2, Wv1, bv1, Wv2, VmL, VmR,
             h_o, x_o, v_o):
    a = acc_r[...]
    sm = a[:, 0] + a[:, 1]
    heL, heR = sm[0], sm[1]
    cxL, cxR = sm[2], sm[3]
    cyL, cyR = sm[4], sm[5]
    czL, czR = sm[6], sm[7]
    svec = sa_r[...] + sb_r[...]
    invc = 1.0 / (svec[:, 4:5] + 1e-10)
    inv2 = invc * invc
    normL = (cxL * cxL + cyL * cyL + czL * czL) * inv2
    normR = (cxR * cxR + cyR * cyR + czR * czR) * inv2
    t1 = normL @ WpnL[...] + normR @ WpnR[...] + bpn1[...]
    t1 = t1 * jax.nn.sigmoid(t1)
    hcomb = t1 @ Wpn2[...] + bpn2[...]
    hcomb = hcomb * jax.nn.sigmoid(hcomb)
    h_ = h_r[...]
    t2 = (h_ @ W1h[...] + heL @ W1eL[...] + heR @ W1eR[...]
          + hcomb @ W1hc[...] + bn1[...])
    t2 = t2 * jax.nn.sigmoid(t2)
    out = t2 @ Wn2[...] + bn2[...]
    out = out * jax.nn.sigmoid(out)
    hn = h_ + out
    dvx = (cxL * invc) @ VmL[...] + (cxR * invc) @ VmR[...]
    dvy = (cyL * invc) @ VmL[...] + (cyR * invc) @ VmR[...]
    dvz = (czL * invc) @ VmL[...] + (czR * invc) @ VmR[...]
    hv = hn @ Wv1[...] + bv1[...]
    hv = hv * jax.nn.sigmoid(hv)
    scale = 2.0 * jax.nn.sigmoid(hv @ Wv2[...])
    ci = lax.broadcasted_iota(jnp.int32, (BN, 16), 1)
    dv = (jnp.where(ci == 0, dvx, 0.0) + jnp.where(ci == 1, dvy, 0.0)
          + jnp.where(ci == 2, dvz, 0.0))
    vn = scale * vp_r[...] + dv
    h_o[...] = hn
    x_o[...] = xp_r[...] + vn
    v_o[...] = vn


def _p6(h, xp, vp, sa, sb, acc, WpnL, WpnR, bpn1, Wpn2, bpn2, W1h, W1eL, W1eR,
        W1hc, bn1, Wn2, bn2, Wv1, bv1, Wv2, VmL, VmR):
    wspec = lambda shp: pl.BlockSpec(shp, lambda i: (0, 0))
    nspec16 = pl.BlockSpec((BN, 16), lambda i: (i, 0))
    return pl.pallas_call(
        _p6_body,
        grid=(N // BN,),
        in_specs=[
            pl.BlockSpec((BN, D), lambda i: (i, 0)),
            nspec16, nspec16, nspec16, nspec16,
            pl.BlockSpec((8, 2, BN, D), lambda i: (0, 0, i, 0)),
            wspec((D, H)), wspec((D, H)), wspec((1, H)), wspec((H, H)),
            wspec((1, H)),
            wspec((D, H)), wspec((D, H)), wspec((D, H)), wspec((H, H)),
            wspec((1, H)), wspec((H, D)), wspec((1, D)),
            wspec((D, H)), wspec((1, H)), wspec((H, 1)),
            wspec((D, 1)), wspec((D, 1)),
        ],
        out_specs=[
            pl.BlockSpec((BN, D), lambda i: (i, 0)),
            nspec16, nspec16,
        ],
        out_shape=(
            jax.ShapeDtypeStruct((N, D), f32),
            jax.ShapeDtypeStruct((N, 16), f32),
            jax.ShapeDtypeStruct((N, 16), f32),
        ),
    )(h, xp, vp, sa, sb, acc, WpnL, WpnR, bpn1, Wpn2, bpn2, W1h, W1eL, W1eR,
      W1hc, bn1, Wn2, bn2, Wv1, bv1, Wv2, VmL, VmR)


def kernel(h, x, v, edges, W_in, b_in, W_out1, b_out1, W_out2, b_out2, W_att,
           b_att, W_xmix, W_pn1, b_pn1, W_pn2, b_pn2, W_node1, b_node1,
           W_node2, b_node2, W_vel1, b_vel1, W_vel2, W_vmix):
    src = edges[:, 0].astype(jnp.int32)
    dst = edges[:, 1].astype(jnp.int32)
    pad = E2 - E
    src2 = jnp.concatenate([src, jnp.zeros((pad,), jnp.int32)]).reshape(
        NCHUNK, CH)
    dst2 = jnp.concatenate([dst, jnp.zeros((pad,), jnp.int32)]).reshape(
        NCHUNK, CH)
    xp = jnp.concatenate([x, jnp.zeros((N, 13), f32)], axis=1)
    vp = jnp.concatenate([v, jnp.zeros((N, 13), f32)], axis=1)
    hb = lax.bitcast_convert_type(
        h.astype(bf16).reshape(N, D // 2, 2), f32)
    hx80 = jnp.concatenate([hb, x, jnp.zeros((N, 13), f32)], axis=1)
    z16 = jnp.zeros((CH, 16), f32)
    z128 = jnp.zeros((CH, D), f32)

    Wina0, Winb0 = W_in[:D], W_in[D:]
    Wina = jnp.concatenate([Wina0[0::2], Wina0[1::2]], axis=0).astype(bf16)
    Winb = jnp.concatenate([Winb0[0::2], Winb0[1::2]], axis=0).astype(bf16)
    bin_ = b_in.reshape(1, K)
    means = jnp.linspace(math.exp(-5.0), 1.0, K,
                         dtype=f32).reshape(1, K)
    W1a0, W1b0 = W_out1[:D], W_out1[D:2 * D]
    W1a = jnp.concatenate([W1a0[0::2], W1a0[1::2]], axis=0).astype(bf16)
    W1b = jnp.concatenate([W1b0[0::2], W1b0[1::2]], axis=0).astype(bf16)
    W1fx = W_out1[2 * D:2 * D + K]
    W1dn = W_out1[2 * D + K:].reshape(1, H)
    bo1 = b_out1.reshape(1, H)
    bo2 = b_out2.reshape(1, H)
    Watt = jnp.concatenate([W_att, jnp.zeros((H, 12), f32)], axis=1)
    batt = jnp.concatenate([b_att, jnp.zeros((12,), f32)]).reshape(1, 16)

    WpnL, WpnR = W_pn1[:D], W_pn1[D:]
    bpn1 = b_pn1.reshape(1, H)
    bpn2 = b_pn2.reshape(1, H)
    W1h = W_node1[:D]
    W1eL = W_node1[D:2 * D]
    W1eR = W_node1[2 * D:3 * D]
    W1hc = W_node1[3 * D:]
    bn1 = b_node1.reshape(1, H)
    bn2 = b_node2.reshape(1, D)
    bv1 = b_vel1.reshape(1, H)
    VmL, VmR = W_vmix[:D], W_vmix[D:]

    hxs, hxd = _p1(hx80, src2, dst2)
    he, wc, xn = _p2(hxs, hxd, Wina, Winb, bin_, means, W1a, W1b,
                     W1fx, W1dn, bo1, W_out2, bo2, Watt, batt)
    sa, sb = _p3(wc, dst2, z16)
    ga, gb = _p3b(sa, sb, dst2)
    edats = _p4(he, wc, xn, ga, gb, W_xmix.astype(bf16))
    acc = _p5(edats, dst2, z128)
    hn, xo, vo = _p6(h, xp, vp, sa, sb, acc, WpnL, WpnR, bpn1, W_pn2, bpn2,
                     W1h, W1eL, W1eR, W1hc, bn1, W_node2, bn2, W_vel1, bv1,
                     W_vel2, VmL, VmR)
    return hn, xo[:, :3], vo[:, :3]


# speedup vs baseline: 1.0225x; 1.0225x over previous
import math

import jax
import jax.numpy as jnp
from jax import lax
from jax.experimental import pallas as pl
from jax.experimental.pallas import tpu as pltpu
from jax.experimental.pallas import tpu_sc as plsc

N = 10000
E = 160000
D = 128
H = 64
NH = 4
NC = 256
K = 50

NWORK = 32
CH = 128
CPW = 40
E2 = NWORK * CPW * CH
NCHUNK = E2 // CH
NTILE = 16
TPN = N // NTILE
TPN_C = 125

BETA = (2.0 / K * (1.0 - math.exp(-5.0))) ** (-2.0)

BE = 1024
BE2 = 1024
BN = 1000

_mesh_cache = []


def _mesh():
    if not _mesh_cache:
        _mesh_cache.append(
            plsc.VectorSubcoreMesh(core_axis_name="c", subcore_axis_name="s"))
    return _mesh_cache[0]


f32 = jnp.float32


bf16 = jnp.bfloat16


def _p1_body(hb_hbm, xp_hbm, src_hbm, dst_hbm, hs_out, hd_out, xs_out, xd_out,
             sidx, didx, hbs0, hbs1, hbd0, hbd1, xbs0, xbs1, xbd0, xbd1,
             g0s, g1s, g2s, g3s, g4s, g5s, g6s, g7s,
             w0s, w1s, w2s, w3s, w4s, w5s, w6s, w7s):
    c = lax.axis_index("c")
    s = lax.axis_index("s")
    wid = s * 2 + c
    g0 = wid * CPW
    pltpu.sync_copy(src_hbm.at[pl.ds(g0, CPW)], sidx)
    pltpu.sync_copy(dst_hbm.at[pl.ds(g0, CPW)], didx)

    def body(j, carry):
        gg = g0 + 2 * j
        c0 = pltpu.async_copy(hb_hbm.at[sidx.at[2 * j]], hbs0, g0s)
        c1 = pltpu.async_copy(hb_hbm.at[didx.at[2 * j]], hbd0, g1s)
        c2 = pltpu.async_copy(xp_hbm.at[sidx.at[2 * j]], xbs0, g2s)
        c3 = pltpu.async_copy(xp_hbm.at[didx.at[2 * j]], xbd0, g3s)
        c4 = pltpu.async_copy(hb_hbm.at[sidx.at[2 * j + 1]], hbs1, g4s)
        c5 = pltpu.async_copy(hb_hbm.at[didx.at[2 * j + 1]], hbd1, g5s)
        c6 = pltpu.async_copy(xp_hbm.at[sidx.at[2 * j + 1]], xbs1, g6s)
        c7 = pltpu.async_copy(xp_hbm.at[didx.at[2 * j + 1]], xbd1, g7s)
        c0.wait()
        w0 = pltpu.async_copy(hbs0, hs_out.at[pl.ds(gg * CH, CH)], w0s)
        c1.wait()
        w1 = pltpu.async_copy(hbd0, hd_out.at[pl.ds(gg * CH, CH)], w1s)
        c2.wait()
        w2 = pltpu.async_copy(xbs0, xs_out.at[pl.ds(gg * CH, CH)], w2s)
        c3.wait()
        w3 = pltpu.async_copy(xbd0, xd_out.at[pl.ds(gg * CH, CH)], w3s)
        c4.wait()
        w4 = pltpu.async_copy(hbs1, hs_out.at[pl.ds((gg + 1) * CH, CH)], w4s)
        c5.wait()
        w5 = pltpu.async_copy(hbd1, hd_out.at[pl.ds((gg + 1) * CH, CH)], w5s)
        c6.wait()
        w6 = pltpu.async_copy(xbs1, xs_out.at[pl.ds((gg + 1) * CH, CH)], w6s)
        c7.wait()
        w7 = pltpu.async_copy(xbd1, xd_out.at[pl.ds((gg + 1) * CH, CH)], w7s)
        w0.wait()
        w1.wait()
        w2.wait()
        w3.wait()
        w4.wait()
        w5.wait()
        w6.wait()
        w7.wait()
        return carry

    lax.fori_loop(0, CPW // 2, body, 0)


def _p1(hb, xp, src2, dst2):
    return pl.kernel(
        _p1_body,
        out_type=(
            jax.ShapeDtypeStruct((E2, D // 2), f32),
            jax.ShapeDtypeStruct((E2, D // 2), f32),
            jax.ShapeDtypeStruct((E2, 16), f32),
            jax.ShapeDtypeStruct((E2, 16), f32),
        ),
        mesh=_mesh(),
        compiler_params=pltpu.CompilerParams(use_tc_tiling_on_sc=False),
        scratch_types=[
            pltpu.VMEM((CPW, CH), jnp.int32),
            pltpu.VMEM((CPW, CH), jnp.int32),
            pltpu.VMEM((CH, D // 2), f32),
            pltpu.VMEM((CH, D // 2), f32),
            pltpu.VMEM((CH, D // 2), f32),
            pltpu.VMEM((CH, D // 2), f32),
            pltpu.VMEM((CH, 16), f32),
            pltpu.VMEM((CH, 16), f32),
            pltpu.VMEM((CH, 16), f32),
            pltpu.VMEM((CH, 16), f32),
        ] + [pltpu.SemaphoreType.DMA] * 16,
    )(hb, xp, src2, dst2)


def _p2_body(hs_r, hd_r, xs_r, xd_r, Wina, Winb, bin_, means, W1a, W1b, W1fx,
             W1dn, bo1, Wo2, bo2, Watt, batt, he_o, wc_o, xn_o):
    i = pl.program_id(0)
    hsu = lax.bitcast_convert_type(hs_r[...], jnp.uint32)
    hdu = lax.bitcast_convert_type(hd_r[...], jnp.uint32)
    hse = lax.bitcast_convert_type(hsu << 16, f32).astype(bf16)
    hso = lax.bitcast_convert_type(hsu & jnp.uint32(0xFFFF0000),
                                   f32).astype(bf16)
    hde = lax.bitcast_convert_type(hdu << 16, f32).astype(bf16)
    hdo = lax.bitcast_convert_type(hdu & jnp.uint32(0xFFFF0000),
                                   f32).astype(bf16)
    dx = xs_r[...] - xd_r[...]
    dn = jnp.sqrt(jnp.sum(dx * dx, axis=-1, keepdims=True) + 1e-14)
    wa, wb = Wina[...], Winb[...]
    h1 = (jnp.dot(hse, wa[:D // 2], preferred_element_type=f32)
          + jnp.dot(hso, wa[D // 2:], preferred_element_type=f32)
          + jnp.dot(hde, wb[:D // 2], preferred_element_type=f32)
          + jnp.dot(hdo, wb[D // 2:], preferred_element_type=f32)
          + bin_[...])
    expn = jnp.exp(-BETA * (jnp.exp(-dn) - means[...]) ** 2)
    fx = expn * h1
    wc1, wd1 = W1a[...], W1b[...]
    t = (jnp.dot(hse, wc1[:D // 2], preferred_element_type=f32)
         + jnp.dot(hso, wc1[D // 2:], preferred_element_type=f32)
         + jnp.dot(hde, wd1[:D // 2], preferred_element_type=f32)
         + jnp.dot(hdo, wd1[D // 2:], preferred_element_type=f32)
         + fx @ W1fx[...] + dn * W1dn[...] + bo1[...])
    t = t * jax.nn.sigmoid(t)
    he = t @ Wo2[...] + bo2[...]
    att = he @ Watt[...] + batt[...]
    cel = jnp.where(att > 0, att, 2.0 * (jnp.exp(att * 0.5) - 1.0))
    w = jnp.exp(cel)
    ci = lax.broadcasted_iota(jnp.int32, (BE, 16), 1)
    rowid = i * BE + lax.broadcasted_iota(jnp.int32, (BE, 1), 0)
    valid = (rowid < E).astype(f32)
    wc = (jnp.where(ci < 4, w, 0.0) + jnp.where(ci == 4, 1.0, 0.0)) * valid
    he_o[...] = he
    wc_o[...] = wc
    xn_o[...] = dx / (dn + 1e-5)


def _p2(hs16, hd16, xs, xd, Wina, Winb, bin_, means, W1a, W1b, W1fx, W1dn,
        bo1, Wo2, bo2, Watt, batt):
    wspec = lambda shp: pl.BlockSpec(shp, lambda i: (0, 0))
    return pl.pallas_call(
        _p2_body,
        grid=(E2 // BE,),
        in_specs=[
            pl.BlockSpec((BE, D // 2), lambda i: (i, 0)),
            pl.BlockSpec((BE, D // 2), lambda i: (i, 0)),
            pl.BlockSpec((BE, 16), lambda i: (i, 0)),
            pl.BlockSpec((BE, 16), lambda i: (i, 0)),
            wspec((D, K)), wspec((D, K)), wspec((1, K)), wspec((1, K)),
            wspec((D, H)), wspec((D, H)), wspec((K, H)), wspec((1, H)),
            wspec((1, H)), wspec((H, H)), wspec((1, H)),
            wspec((H, 16)), wspec((1, 16)),
        ],
        out_specs=[
            pl.BlockSpec((BE, H), lambda i: (i, 0)),
            pl.BlockSpec((BE, 16), lambda i: (i, 0)),
            pl.BlockSpec((BE, 16), lambda i: (i, 0)),
        ],
        out_shape=(
            jax.ShapeDtypeStruct((E2, H), f32),
            jax.ShapeDtypeStruct((E2, 16), f32),
            jax.ShapeDtypeStruct((E2, 16), f32),
        ),
    )(hs16, hd16, xs, xd, Wina, Winb, bin_, means, W1a, W1b, W1fx, W1dn, bo1,
      Wo2, bo2, Watt, batt)


def _p3_body(wc_hbm, dst_hbm, z16_hbm, sa_out, sb_out, acc, idx, wbuf, zv):
    c = lax.axis_index("c")
    s = lax.axis_index("s")
    pltpu.sync_copy(z16_hbm, zv)
    r0 = s * TPN
    for k in range(5):
        pltpu.sync_copy(zv.at[pl.ds(0, TPN_C)],
                        acc.at[pl.ds(r0 + k * TPN_C, TPN_C)])
    plsc.subcore_barrier()
    g0 = (c * NTILE + s) * CPW
    pltpu.sync_copy(dst_hbm.at[pl.ds(g0, CPW)], idx)

    def body(g, carry):
        row0 = (g0 + g) * CH
        pltpu.sync_copy(wc_hbm.at[pl.ds(row0, CH)], wbuf)
        pltpu.sync_copy(wbuf, acc.at[idx.at[g]], add=True)
        return carry

    lax.fori_loop(0, CPW, body, 0)
    plsc.subcore_barrier()
    for k in range(5):
        rr = r0 + k * TPN_C

        @pl.when(c == 0)
        def _():
            pltpu.sync_copy(acc.at[pl.ds(rr, TPN_C)],
                            sa_out.at[pl.ds(rr, TPN_C)])

        @pl.when(c == 1)
        def _():
            pltpu.sync_copy(acc.at[pl.ds(rr, TPN_C)],
                            sb_out.at[pl.ds(rr, TPN_C)])


def _p3(wc, dst2, z16):
    return pl.kernel(
        _p3_body,
        out_type=(
            jax.ShapeDtypeStruct((N, 16), f32),
            jax.ShapeDtypeStruct((N, 16), f32),
        ),
        mesh=_mesh(),
        compiler_params=pltpu.CompilerParams(use_tc_tiling_on_sc=False),
        scratch_types=[
            pltpu.VMEM_SHARED((N, 16), f32),
            pltpu.VMEM((CPW, CH), jnp.int32),
            pltpu.VMEM((CH, 16), f32),
            pltpu.VMEM((CH, 16), f32),
        ],
    )(wc, dst2, z16)


def _p3b_body(sa_hbm, sb_hbm, dst_hbm, ga_out, gb_out, idx, a0, b0, a1, b1,
              ga0, gb0, ga1, gb1, wa0, wb0, wa1, wb1):
    c = lax.axis_index("c")
    s = lax.axis_index("s")
    wid = s * 2 + c
    g0 = wid * CPW
    pltpu.sync_copy(dst_hbm.at[pl.ds(g0, CPW)], idx)

    def body(j, carry):
        gg = g0 + 2 * j
        cpa0 = pltpu.async_copy(sa_hbm.at[idx.at[2 * j]], a0, ga0)
        cpb0 = pltpu.async_copy(sb_hbm.at[idx.at[2 * j]], b0, gb0)
        cpa1 = pltpu.async_copy(sa_hbm.at[idx.at[2 * j + 1]], a1, ga1)
        cpb1 = pltpu.async_copy(sb_hbm.at[idx.at[2 * j + 1]], b1, gb1)
        cpa0.wait()
        w0 = pltpu.async_copy(a0, ga_out.at[pl.ds(gg * CH, CH)], wa0)
        cpb0.wait()
        w1 = pltpu.async_copy(b0, gb_out.at[pl.ds(gg * CH, CH)], wb0)
        cpa1.wait()
        w2 = pltpu.async_copy(a1, ga_out.at[pl.ds((gg + 1) * CH, CH)], wa1)
        cpb1.wait()
        w3 = pltpu.async_copy(b1, gb_out.at[pl.ds((gg + 1) * CH, CH)], wb1)
        w0.wait()
        w1.wait()
        w2.wait()
        w3.wait()
        return carry

    lax.fori_loop(0, CPW // 2, body, 0)


def _p3b(sa, sb, dst2):
    return pl.kernel(
        _p3b_body,
        out_type=(
            jax.ShapeDtypeStruct((E2, 16), f32),
            jax.ShapeDtypeStruct((E2, 16), f32),
        ),
        mesh=_mesh(),
        compiler_params=pltpu.CompilerParams(use_tc_tiling_on_sc=False),
        scratch_types=[
            pltpu.VMEM((CPW, CH), jnp.int32),
            pltpu.VMEM((CH, 16), f32),
            pltpu.VMEM((CH, 16), f32),
            pltpu.VMEM((CH, 16), f32),
            pltpu.VMEM((CH, 16), f32),
        ] + [pltpu.SemaphoreType.DMA] * 8,
    )(sa, sb, dst2)


def _p4_body(he_r, wc_r, xn_r, ga_r, gb_r, Wx, o0, o1, o2, o3, o4, o5, o6, o7):
    i = pl.program_id(0)
    s = ga_r[...] + gb_r[...]
    catt = wc_r[...][:, 0:4] / (s[:, 0:4] + 1e-16)
    he = he_r[...]
    hea = jnp.concatenate([he * catt[:, k:k + 1] for k in range(4)], axis=1)
    coeff = jnp.tanh(jnp.dot(hea.astype(bf16), Wx[...],
                             preferred_element_type=f32))
    rowid = i * BE2 + lax.broadcasted_iota(jnp.int32, (BE2, 1), 0)
    valid = (rowid < E).astype(f32)
    xn = xn_r[...]
    heam = hea * valid
    cx = coeff * xn[:, 0:1] * valid
    cy = coeff * xn[:, 1:2] * valid
    cz = coeff * xn[:, 2:3] * valid
    o0[...] = heam[:, :D]
    o1[...] = heam[:, D:]
    o2[...] = cx[:, :D]
    o3[...] = cx[:, D:]
    o4[...] = cy[:, :D]
    o5[...] = cy[:, D:]
    o6[...] = cz[:, :D]
    o7[...] = cz[:, D:]


def _p4(he, wc, xn, ga, gb, Wx):
    espec = pl.BlockSpec((BE2, 16), lambda i: (i, 0))
    ospec = pl.BlockSpec((BE2, D), lambda i: (i, 0))
    return pl.pallas_call(
        _p4_body,
        grid=(E2 // BE2,),
        in_specs=[
            pl.BlockSpec((BE2, H), lambda i: (i, 0)),
            espec, espec, espec, espec,
            pl.BlockSpec((NC, NC), lambda i: (0, 0)),
        ],
        out_specs=[ospec] * 8,
        out_shape=tuple(jax.ShapeDtypeStruct((E2, D), f32) for _ in range(8)),
    )(he, wc, xn, ga, gb, Wx)


def _p5_body(e0, e1, e2, e3, e4, e5, e6, e7, dst_hbm, z128_hbm, out_hbm,
             acc, idx, db0, db1, l0, l1, s0, s1):
    c = lax.axis_index("c")
    s = lax.axis_index("s")
    g0 = (c * NTILE + s) * CPW
    pltpu.sync_copy(dst_hbm.at[pl.ds(g0, CPW)], idx)
    r0 = s * TPN
    for p, e_hbm in enumerate([e0, e1, e2, e3, e4, e5, e6, e7]):
        for k in range(5):
            pltpu.sync_copy(z128_hbm.at[pl.ds(0, TPN_C)],
                            acc.at[pl.ds(r0 + k * TPN_C, TPN_C)])
        plsc.subcore_barrier()

        def body(j, carry):
            gg = g0 + 2 * j
            cl0 = pltpu.async_copy(e_hbm.at[pl.ds(gg * CH, CH)], db0, l0)
            cl1 = pltpu.async_copy(e_hbm.at[pl.ds((gg + 1) * CH, CH)], db1, l1)
            cl0.wait()
            cs0 = pltpu.async_copy(db0, acc.at[idx.at[2 * j]], s0, add=True)
            cl1.wait()
            cs1 = pltpu.async_copy(db1, acc.at[idx.at[2 * j + 1]], s1,
                                   add=True)
            cs0.wait()
            cs1.wait()
            return carry

        lax.fori_loop(0, CPW // 2, body, 0)
        plsc.subcore_barrier()
        for k in range(5):
            rr = r0 + k * TPN_C
            pltpu.sync_copy(acc.at[pl.ds(rr, TPN_C)],
                            out_hbm.at[p, c, pl.ds(rr, TPN_C)])
        plsc.subcore_barrier()


def _p5(edats, dst2, z128):
    return pl.kernel(
        _p5_body,
        out_type=jax.ShapeDtypeStruct((8, 2, N, D), f32),
        mesh=_mesh(),
        compiler_params=pltpu.CompilerParams(use_tc_tiling_on_sc=False),
        scratch_types=[
            pltpu.VMEM_SHARED((N, D), f32),
            pltpu.VMEM((CPW, CH), jnp.int32),
            pltpu.VMEM((CH, D), f32),
            pltpu.VMEM((CH, D), f32),
        ] + [pltpu.SemaphoreType.DMA] * 4,
    )(*edats, dst2, z128)


def _p6_body(h_r, xp_r, vp_r, sa_r, sb_r, acc_r, WpnL, WpnR, bpn1, Wpn2, bpn2,
             W1h, W1eL, W1eR, W1hc, bn1, Wn2, bn2, Wv1, bv1, Wv2, VmL, VmR,
             h_o, x_o, v_o):
    a = acc_r[...]
    sm = a[:, 0] + a[:, 1]
    heL, heR = sm[0], sm[1]
    cxL, cxR = sm[2], sm[3]
    cyL, cyR = sm[4], sm[5]
    czL, czR = sm[6], sm[7]
    svec = sa_r[...] + sb_r[...]
    invc = 1.0 / (svec[:, 4:5] + 1e-10)
    inv2 = invc * invc
    normL = (cxL * cxL + cyL * cyL + czL * czL) * inv2
    normR = (cxR * cxR + cyR * cyR + czR * czR) * inv2
    t1 = normL @ WpnL[...] + normR @ WpnR[...] + bpn1[...]
    t1 = t1 * jax.nn.sigmoid(t1)
    hcomb = t1 @ Wpn2[...] + bpn2[...]
    hcomb = hcomb * jax.nn.sigmoid(hcomb)
    h_ = h_r[...]
    t2 = (h_ @ W1h[...] + heL @ W1eL[...] + heR @ W1eR[...]
          + hcomb @ W1hc[...] + bn1[...])
    t2 = t2 * jax.nn.sigmoid(t2)
    out = t2 @ Wn2[...] + bn2[...]
    out = out * jax.nn.sigmoid(out)
    hn = h_ + out
    dvx = (cxL * invc) @ VmL[...] + (cxR * invc) @ VmR[...]
    dvy = (cyL * invc) @ VmL[...] + (cyR * invc) @ VmR[...]
    dvz = (czL * invc) @ VmL[...] + (czR * invc) @ VmR[...]
    hv = hn @ Wv1[...] + bv1[...]
    hv = hv * jax.nn.sigmoid(hv)
    scale = 2.0 * jax.nn.sigmoid(hv @ Wv2[...])
    ci = lax.broadcasted_iota(jnp.int32, (BN, 16), 1)
    dv = (jnp.where(ci == 0, dvx, 0.0) + jnp.where(ci == 1, dvy, 0.0)
          + jnp.where(ci == 2, dvz, 0.0))
    vn = scale * vp_r[...] + dv
    h_o[...] = hn
    x_o[...] = xp_r[...] + vn
    v_o[...] = vn


def _p6(h, xp, vp, sa, sb, acc, WpnL, WpnR, bpn1, Wpn2, bpn2, W1h, W1eL, W1eR,
        W1hc, bn1, Wn2, bn2, Wv1, bv1, Wv2, VmL, VmR):
    wspec = lambda shp: pl.BlockSpec(shp, lambda i: (0, 0))
    nspec16 = pl.BlockSpec((BN, 16), lambda i: (i, 0))
    return pl.pallas_call(
        _p6_body,
        grid=(N // BN,),
        in_specs=[
            pl.BlockSpec((BN, D), lambda i: (i, 0)),
            nspec16, nspec16, nspec16, nspec16,
            pl.BlockSpec((8, 2, BN, D), lambda i: (0, 0, i, 0)),
            wspec((D, H)), wspec((D, H)), wspec((1, H)), wspec((H, H)),
            wspec((1, H)),
            wspec((D, H)), wspec((D, H)), wspec((D, H)), wspec((H, H)),
            wspec((1, H)), wspec((H, D)), wspec((1, D)),
            wspec((D, H)), wspec((1, H)), wspec((H, 1)),
            wspec((D, 1)), wspec((D, 1)),
        ],
        out_specs=[
            pl.BlockSpec((BN, D), lambda i: (i, 0)),
            nspec16, nspec16,
        ],
        out_shape=(
            jax.ShapeDtypeStruct((N, D), f32),
            jax.ShapeDtypeStruct((N, 16), f32),
            jax.ShapeDtypeStruct((N, 16), f32),
        ),
    )(h, xp, vp, sa, sb, acc, WpnL, WpnR, bpn1, Wpn2, bpn2, W1h, W1eL, W1eR,
      W1hc, bn1, Wn2, bn2, Wv1, bv1, Wv2, VmL, VmR)


def kernel(h, x, v, edges, W_in, b_in, W_out1, b_out1, W_out2, b_out2, W_att,
           b_att, W_xmix, W_pn1, b_pn1, W_pn2, b_pn2, W_node1, b_node1,
           W_node2, b_node2, W_vel1, b_vel1, W_vel2, W_vmix):
    src = edges[:, 0].astype(jnp.int32)
    dst = edges[:, 1].astype(jnp.int32)
    pad = E2 - E
    src2 = jnp.concatenate([src, jnp.zeros((pad,), jnp.int32)]).reshape(
        NCHUNK, CH)
    dst2 = jnp.concatenate([dst, jnp.zeros((pad,), jnp.int32)]).reshape(
        NCHUNK, CH)
    xp = jnp.concatenate([x, jnp.zeros((N, 13), f32)], axis=1)
    vp = jnp.concatenate([v, jnp.zeros((N, 13), f32)], axis=1)
    hb = lax.bitcast_convert_type(
        h.astype(bf16).reshape(N, D // 2, 2), f32)
    z16 = jnp.zeros((CH, 16), f32)
    z128 = jnp.zeros((CH, D), f32)

    Wina0, Winb0 = W_in[:D], W_in[D:]
    Wina = jnp.concatenate([Wina0[0::2], Wina0[1::2]], axis=0).astype(bf16)
    Winb = jnp.concatenate([Winb0[0::2], Winb0[1::2]], axis=0).astype(bf16)
    bin_ = b_in.reshape(1, K)
    means = jnp.linspace(math.exp(-5.0), 1.0, K,
                         dtype=f32).reshape(1, K)
    W1a0, W1b0 = W_out1[:D], W_out1[D:2 * D]
    W1a = jnp.concatenate([W1a0[0::2], W1a0[1::2]], axis=0).astype(bf16)
    W1b = jnp.concatenate([W1b0[0::2], W1b0[1::2]], axis=0).astype(bf16)
    W1fx = W_out1[2 * D:2 * D + K]
    W1dn = W_out1[2 * D + K:].reshape(1, H)
    bo1 = b_out1.reshape(1, H)
    bo2 = b_out2.reshape(1, H)
    Watt = jnp.concatenate([W_att, jnp.zeros((H, 12), f32)], axis=1)
    batt = jnp.concatenate([b_att, jnp.zeros((12,), f32)]).reshape(1, 16)

    WpnL, WpnR = W_pn1[:D], W_pn1[D:]
    bpn1 = b_pn1.reshape(1, H)
    bpn2 = b_pn2.reshape(1, H)
    W1h = W_node1[:D]
    W1eL = W_node1[D:2 * D]
    W1eR = W_node1[2 * D:3 * D]
    W1hc = W_node1[3 * D:]
    bn1 = b_node1.reshape(1, H)
    bn2 = b_node2.reshape(1, D)
    bv1 = b_vel1.reshape(1, H)
    VmL, VmR = W_vmix[:D], W_vmix[D:]

    hs16, hd16, xs, xd = _p1(hb, xp, src2, dst2)
    he, wc, xn = _p2(hs16, hd16, xs, xd, Wina, Winb, bin_, means, W1a, W1b,
                     W1fx, W1dn, bo1, W_out2, bo2, Watt, batt)
    sa, sb = _p3(wc, dst2, z16)
    ga, gb = _p3b(sa, sb, dst2)
    edats = _p4(he, wc, xn, ga, gb, W_xmix.astype(bf16))
    acc = _p5(edats, dst2, z128)
    hn, xo, vo = _p6(h, xp, vp, sa, sb, acc, WpnL, WpnR, bpn1, W_pn2, bpn2,
                     W1h, W1eL, W1eR, W1hc, bn1, W_node2, bn2, W_vel1, bv1,
                     W_vel2, VmL, VmR)
    return hn, xo[:, :3], vo[:, :3]
